# baseline (speedup 1.0000x reference)
import jax
import jax.numpy as jnp
from jax import lax
from jax.experimental import pallas as pl
from jax.experimental.pallas import tpu as pltpu
from jax.experimental.pallas import tpu_sc as plsc

_EMB_DIM = 128
_BATCH = 4096
_PAD = 100352
_NSUB = 16
_CHUNK = 128
_NCHUNK = _BATCH // (_NSUB * _CHUNK)
_SLICE = _PAD // _NSUB
_GROUPS = _EMB_DIM // 16


def _sc_body(posI, negI, ent, rel,
             main_o, esum_o, ecnt_o, rsum_o, rcnt_o,
             idx_p, idx_n,
             hp, rp, tp, hn, rn, tn,
             vh_p, vt_p, vr_p, vh_n, vt_n,
             ones_v, mbuf, zbuf,
             esum_s, ecnt_s, rsum_s, rcnt_s,
             sem_g, sem_i, sem_s):
    s = lax.axis_index("s")

    zero16 = jnp.zeros((16,), jnp.float32)
    one16 = jnp.ones((16,), jnp.float32)
    lane = lax.iota(jnp.int32, 16)
    last = lane == 15

    def zfill(i, carry):
        zbuf[pl.ds(i * 16, 16)] = zero16
        return carry

    lax.fori_loop(0, _SLICE // 64, zfill, None)
    off = s * _SLICE
    inits = []
    for arr in (esum_s, ecnt_s, rsum_s, rcnt_s):
        for q in range(4):
            inits.append(pltpu.async_copy(
                zbuf, arr.at[pl.ds(off + q * (_SLICE // 4), _SLICE // 4)],
                sem_i))
    for i in range(_CHUNK // 16):
        ones_v[pl.ds(i * 16, 16)] = one16

    gdn = lax.GatherDimensionNumbers(
        offset_dims=(), collapsed_slice_dims=(0,), start_index_map=(0,))

    def hsum(x):
        for k in (1, 2, 4, 8):
            perm = lax.gather(x, (lane ^ k)[:, None], gdn, slice_sizes=(1,),
                              mode=lax.GatherScatterMode.PROMISE_IN_BOUNDS)
            x = x + perm
        return x

    def vsqrt(x):
        i = plsc.bitcast(x, jnp.int32)
        y = plsc.bitcast(jnp.int32(0x1FBD1DF5) + (i >> 1), jnp.float32)
        for _ in range(3):
            y = 0.5 * (y + x / y)
        return y

    def edge_body(e, macc):
        dps = dns = hps = tps = rps = hns = tns = zero16
        for j in range(_GROUPS):
            col = pl.ds(j * 16, 16)
            hpv = hp[e, col]
            rpv = rp[e, col]
            tpv = tp[e, col]
            hnv = hn[e, col]
            rnv = rn[e, col]
            tnv = tn[e, col]
            dp = hpv + rpv - tpv
            dn = hnv + rnv - tnv
            dps = dps + dp * dp
            dns = dns + dn * dn
            hps = hps + hpv * hpv
            tps = tps + tpv * tpv
            rps = rps + rpv * rpv
            hns = hns + hnv * hnv
            tns = tns + tnv * tnv
        eidx = jnp.full((16,), e, jnp.int32)
        plsc.store_scatter(vh_p, [eidx], hsum(hps), mask=last)
        plsc.store_scatter(vt_p, [eidx], hsum(tps), mask=last)
        plsc.store_scatter(vr_p, [eidx], hsum(rps), mask=last)
        plsc.store_scatter(vh_n, [eidx], hsum(hns), mask=last)
        plsc.store_scatter(vt_n, [eidx], hsum(tns), mask=last)
        contrib = jnp.maximum(1.0 + vsqrt(hsum(dps)) - vsqrt(hsum(dns)), 0.0)
        return macc + jnp.where(last, contrib, 0.0)

    macc = zero16
    barrier_done = False
    for chunk in range(_NCHUNK):
        blk = s * _NCHUNK + chunk
        pltpu.sync_copy(posI.at[blk], idx_p)
        pltpu.sync_copy(negI.at[blk], idx_n)
        gathers = [
            pltpu.async_copy(ent.at[idx_p.at[0]], hp, sem_g),
            pltpu.async_copy(rel.at[idx_p.at[1]], rp, sem_g),
            pltpu.async_copy(ent.at[idx_p.at[2]], tp, sem_g),
            pltpu.async_copy(ent.at[idx_n.at[0]], hn, sem_g),
            pltpu.async_copy(rel.at[idx_n.at[1]], rn, sem_g),
            pltpu.async_copy(ent.at[idx_n.at[2]], tn, sem_g),
        ]
        for g in gathers:
            g.wait()
        if not barrier_done:
            for i in inits:
                i.wait()
        macc = lax.fori_loop(0, _CHUNK, edge_body, macc)
        if not barrier_done:
            plsc.subcore_barrier()
            barrier_done = True
        scatters = [
            pltpu.async_copy(vh_p, esum_s.at[idx_p.at[0]], sem_s, add=True),
            pltpu.async_copy(ones_v, ecnt_s.at[idx_p.at[0]], sem_s, add=True),
            pltpu.async_copy(vt_p, esum_s.at[idx_p.at[2]], sem_s, add=True),
            pltpu.async_copy(ones_v, ecnt_s.at[idx_p.at[2]], sem_s, add=True),
            pltpu.async_copy(vh_n, esum_s.at[idx_n.at[0]], sem_s, add=True),
            pltpu.async_copy(ones_v, ecnt_s.at[idx_n.at[0]], sem_s, add=True),
            pltpu.async_copy(vt_n, esum_s.at[idx_n.at[2]], sem_s, add=True),
            pltpu.async_copy(ones_v, ecnt_s.at[idx_n.at[2]], sem_s, add=True),
            pltpu.async_copy(vr_p, rsum_s.at[idx_p.at[1]], sem_s, add=True),
            pltpu.async_copy(ones_v, rcnt_s.at[idx_p.at[1]], sem_s, add=True),
        ]
        for sd in scatters:
            sd.wait()

    mbuf[pl.ds(0, 16)] = macc
    m0 = pltpu.async_copy(mbuf, main_o.at[s], sem_i)
    m0.wait()

    plsc.subcore_barrier()

    pltpu.sync_copy(esum_s.at[pl.ds(off, _SLICE)], esum_o.at[pl.ds(off, _SLICE)])
    pltpu.sync_copy(ecnt_s.at[pl.ds(off, _SLICE)], ecnt_o.at[pl.ds(off, _SLICE)])
    pltpu.sync_copy(rsum_s.at[pl.ds(off, _SLICE)], rsum_o.at[pl.ds(off, _SLICE)])
    pltpu.sync_copy(rcnt_s.at[pl.ds(off, _SLICE)], rcnt_o.at[pl.ds(off, _SLICE)])


def _tc_reduce(mo, es, ec, rs, rc, out):
    main = jnp.sum(mo[...])

    def scale_loss(sum_ref, cnt_ref):
        tot = sum_ref[...]
        cnt = cnt_ref[...]
        pres = cnt > 0.5
        val = jnp.sqrt(tot / jnp.maximum(cnt, 1.0)) - 1.0
        num = jnp.sum(jnp.where(pres, jnp.maximum(val, 0.0), 0.0))
        den = jnp.sum(jnp.where(pres, 1.0, 0.0))
        return num / den

    total = main + scale_loss(es, ec) + scale_loss(rs, rc)
    out[...] = jnp.reshape(total, (1, 1))


@jax.jit
def _impl(pos_edge, neg_edge, entity_emb, relation_emb):
    nblk = _NSUB * _NCHUNK
    posI = jnp.asarray(pos_edge, jnp.int32).T.reshape(3, nblk, _CHUNK)
    posI = posI.transpose(1, 0, 2)
    negI = jnp.asarray(neg_edge, jnp.int32).T.reshape(3, nblk, _CHUNK)
    negI = negI.transpose(1, 0, 2)

    mesh = plsc.VectorSubcoreMesh(
        core_axis_name="c", subcore_axis_name="s", num_cores=1)
    f32 = jnp.float32
    sc = pl.kernel(
        _sc_body,
        out_type=[
            jax.ShapeDtypeStruct((_NSUB, 16), f32),
            jax.ShapeDtypeStruct((_PAD,), f32),
            jax.ShapeDtypeStruct((_PAD,), f32),
            jax.ShapeDtypeStruct((_PAD,), f32),
            jax.ShapeDtypeStruct((_PAD,), f32),
        ],
        mesh=mesh,
        compiler_params=pltpu.CompilerParams(needs_layout_passes=False),
        scratch_types=[
            pltpu.VMEM((3, _CHUNK), jnp.int32),
            pltpu.VMEM((3, _CHUNK), jnp.int32),
            pltpu.VMEM((_CHUNK, _EMB_DIM), f32),
            pltpu.VMEM((_CHUNK, _EMB_DIM), f32),
            pltpu.VMEM((_CHUNK, _EMB_DIM), f32),
            pltpu.VMEM((_CHUNK, _EMB_DIM), f32),
            pltpu.VMEM((_CHUNK, _EMB_DIM), f32),
            pltpu.VMEM((_CHUNK, _EMB_DIM), f32),
            pltpu.VMEM((_CHUNK,), f32),
            pltpu.VMEM((_CHUNK,), f32),
            pltpu.VMEM((_CHUNK,), f32),
            pltpu.VMEM((_CHUNK,), f32),
            pltpu.VMEM((_CHUNK,), f32),
            pltpu.VMEM((_CHUNK,), f32),
            pltpu.VMEM((16,), f32),
            pltpu.VMEM((_SLICE // 4,), f32),
            pltpu.VMEM_SHARED((_PAD,), f32),
            pltpu.VMEM_SHARED((_PAD,), f32),
            pltpu.VMEM_SHARED((_PAD,), f32),
            pltpu.VMEM_SHARED((_PAD,), f32),
            pltpu.SemaphoreType.DMA,
            pltpu.SemaphoreType.DMA,
            pltpu.SemaphoreType.DMA,
        ],
    )
    mo, es, ec, rs, rc = sc(posI, negI, entity_emb, relation_emb)

    red = pl.pallas_call(
        _tc_reduce,
        out_shape=jax.ShapeDtypeStruct((1, 1), f32),
    )
    loss = red(
        mo,
        es.reshape(_PAD // 128, 128), ec.reshape(_PAD // 128, 128),
        rs.reshape(_PAD // 128, 128), rc.reshape(_PAD // 128, 128),
    )
    return jnp.reshape(loss, ())


def kernel(pos_edge, neg_edge, entity_emb, relation_emb):
    return _impl(pos_edge, neg_edge, entity_emb, relation_emb)
